# Initial kernel scaffold; baseline (speedup 1.0000x reference)
import functools

import jax
import jax.numpy as jnp
import numpy as np
from jax import lax
from jax.experimental import pallas as pl
from jax.experimental.pallas import tpu as pltpu
from jax.experimental.pallas import tpu_sc as plsc

_BN = 1.0 / np.sqrt(1.0 + 1e-5)


def _fps_body(npoint, sub, lanes, has_tbl, *refs):
    if has_tbl:
        xs, ys, zs, tbl, inds, fp2, nx, ny, nz, dists = refs
    else:
        xs, ys, zs, inds, nx, ny, nz, dists = refs
        tbl = fp2 = None
    B = xs.shape[0]
    iota = (lax.broadcasted_iota(jnp.int32, (sub, lanes), 0) * lanes
            + lax.broadcasted_iota(jnp.int32, (sub, lanes), 1))

    dists[...] = jnp.full((B, sub, lanes), 1e10, jnp.float32)

    def step(i, carry):
        nxt = []
        for b in range(B):
            f = carry[b]
            fs = f // lanes
            fl = f - fs * lanes
            cx = xs[b, fs, pl.ds(fl, 1)]
            cy = ys[b, fs, pl.ds(fl, 1)]
            cz = zs[b, fs, pl.ds(fl, 1)]
            inds[b, pl.ds(i, 1)] = jnp.full((1,), f, jnp.int32)
            nx[b, pl.ds(i, 1)] = cx
            ny[b, pl.ds(i, 1)] = cy
            nz[b, pl.ds(i, 1)] = cz
            if has_tbl:
                fp2[b, pl.ds(i, 1)] = tbl[b, pl.ds(f, 1)]
            dx = xs[b] - cx[0]
            dy = ys[b] - cy[0]
            dz = zs[b] - cz[0]
            d = (dx * dx + dy * dy) + dz * dz
            dmin = jnp.minimum(dists[b], d)
            dists[b] = dmin
            m = jnp.max(dmin)
            cand = jnp.where(dmin == m, iota, sub * lanes)
            nxt.append(jnp.min(cand))
        return tuple(nxt)

    lax.fori_loop(0, npoint, step, tuple(jnp.int32(0) for _ in range(B)),
                  unroll=False)


def _fps(xyz, npoint, tbl=None):
    B, N, _ = xyz.shape
    lanes = 1024 if N % 1024 == 0 else 512
    sub = N // lanes
    xs = xyz[:, :, 0].reshape(B, sub, lanes)
    ys = xyz[:, :, 1].reshape(B, sub, lanes)
    zs = xyz[:, :, 2].reshape(B, sub, lanes)
    out_types = [jax.ShapeDtypeStruct((B, npoint), jnp.int32)]
    if tbl is not None:
        out_types.append(jax.ShapeDtypeStruct((B, npoint), jnp.int32))
    out_types += [jax.ShapeDtypeStruct((B, npoint), jnp.float32)] * 3
    args = (xs, ys, zs) + (() if tbl is None else (tbl,))
    outs = pl.pallas_call(
        functools.partial(_fps_body, npoint, sub, lanes, tbl is not None),
        out_shape=tuple(out_types),
        scratch_shapes=[pltpu.VMEM((B, sub, lanes), jnp.float32)],
    )(*args)
    if tbl is not None:
        inds, fp2, nx, ny, nz = outs
        return inds, fp2, nx, ny, nz
    inds, nx, ny, nz = outs
    return inds, None, nx, ny, nz


def _bq_body(nsample, r2, N, qx, qy, qz, xs, ys, zs, idx_out, val):
    b = pl.program_id(0)
    BM = qx.shape[1]
    qxc = jnp.reshape(qx[0, :], (BM, 1))
    qyc = jnp.reshape(qy[0, :], (BM, 1))
    qzc = jnp.reshape(qz[0, :], (BM, 1))
    xr = jnp.reshape(xs[0, :], (1, N))
    yr = jnp.reshape(ys[0, :], (1, N))
    zr = jnp.reshape(zs[0, :], (1, N))
    qsq = qxc * qxc + qyc * qyc + qzc * qzc
    xsq = xr * xr + yr * yr + zr * zr
    dot = qxc * xr + qyc * yr + qzc * zr
    D = jnp.maximum(qsq - 2.0 * dot + xsq, 0.0)
    iota = lax.broadcasted_iota(jnp.int32, (BM, N), 1)
    val[...] = jnp.where(D < r2, iota, N)

    v = val[...]
    m0 = jnp.min(v, axis=1, keepdims=True)
    idx_out[0] = jnp.broadcast_to(m0 + b * N, (BM, nsample)).astype(jnp.int32)
    val[...] = jnp.where(v == m0, N, v)

    def cond(c):
        s, active = c
        return jnp.logical_and(s < nsample, active)

    def body(c):
        s, _ = c
        v = val[...]
        m = jnp.min(v, axis=1, keepdims=True)
        live = m < N
        rec = jnp.where(live, m + b * N, idx_out[0, :, pl.ds(0, 1)])
        idx_out[0, :, pl.ds(s, 1)] = rec.astype(jnp.int32)
        val[...] = jnp.where(v == m, N, v)
        return s + 1, jnp.any(live)

    lax.while_loop(cond, body, (jnp.int32(1), jnp.bool_(True)))


def _ball_query(new_xyz, xyz, radius, nsample, bm):
    B, M, _ = new_xyz.shape
    N = xyz.shape[1]
    qx, qy, qz = (new_xyz[:, :, i] for i in range(3))
    xs, ys, zs = (xyz[:, :, i] for i in range(3))
    grid = (B, M // bm)
    qspec = pl.BlockSpec((1, bm), lambda b, i: (b, i))
    xspec = pl.BlockSpec((1, N), lambda b, i: (b, 0))
    return pl.pallas_call(
        functools.partial(_bq_body, nsample, radius * radius, N),
        grid=grid,
        in_specs=[qspec] * 3 + [xspec] * 3,
        out_specs=pl.BlockSpec((1, bm, nsample), lambda b, i: (b, i, 0)),
        out_shape=jax.ShapeDtypeStruct((B, M, nsample), jnp.int32),
        scratch_shapes=[pltpu.VMEM((bm, N), jnp.int32)],
    )(qx, qy, qz, xs, ys, zs)


def _sc_gather(table, idx2d):
    D = table.shape[1]
    R = idx2d.shape[0] * 128
    NW = 32
    chunks_w = R // NW // 128
    mesh = plsc.VectorSubcoreMesh(core_axis_name="c", subcore_axis_name="s")

    @functools.partial(
        pl.kernel, mesh=mesh,
        out_type=jax.ShapeDtypeStruct((R, D), jnp.float32),
        scratch_types=[
            pltpu.VMEM((chunks_w, 128), jnp.int32),
            pltpu.VMEM((128, D), jnp.float32),
            pltpu.VMEM((128, D), jnp.float32),
            pltpu.SemaphoreType.DMA,
            pltpu.SemaphoreType.DMA,
        ],
    )
    def k(table_hbm, idx_hbm, out_hbm, idx_v, rows0, rows1, sem0, sem1):
        wid = lax.axis_index("s") * 2 + lax.axis_index("c")
        cbase = wid * chunks_w
        pltpu.sync_copy(idx_hbm.at[pl.ds(cbase, chunks_w)], idx_v)
        rows = (rows0, rows1)
        sems = (sem0, sem1)
        pltpu.async_copy(table_hbm.at[idx_v.at[0]], rows0, sem0)

        def body(c, _):
            cur = lax.rem(c, 2)
            for par in range(2):
                @pl.when(cur == par)
                def _():
                    nxt = 1 - par

                    @pl.when(c + 1 < chunks_w)
                    def _():
                        pltpu.async_copy(
                            table_hbm.at[idx_v.at[c + 1]], rows[nxt],
                            sems[nxt])

                    pltpu.make_async_copy(
                        table_hbm.at[idx_v.at[c]], rows[par],
                        sems[par]).wait()
                    pltpu.sync_copy(
                        rows[par], out_hbm.at[pl.ds((cbase + c) * 128, 128)])
            return 0

        lax.fori_loop(0, chunks_w, body, 0, unroll=False)

    return k(table, idx2d)


def _sa_mlp_body(S, g, q, srow, w1, w2, w3, out):
    BMM = q.shape[0]
    C = g.shape[1]
    qrep = jnp.reshape(
        jnp.broadcast_to(q[...][:, None, :], (BMM, S, C)), (BMM * S, C))
    x = (g[...] - qrep) * srow[...]
    x = jnp.maximum(jnp.dot(x, w1[...], preferred_element_type=jnp.float32),
                    0.0)
    x = jnp.maximum(jnp.dot(x, w2[...], preferred_element_type=jnp.float32),
                    0.0)
    x = jnp.maximum(jnp.dot(x, w3[...], preferred_element_type=jnp.float32),
                    0.0)
    out[...] = jnp.max(jnp.reshape(x, (BMM, S, out.shape[1])), axis=1)


def _sa_mlp(gathered, qpad, srow, w1t, w2t, w3t, S, bmm):
    RQ, C = qpad.shape
    Cout = w3t.shape[1]
    grid = (RQ // bmm,)
    full = lambda shape: pl.BlockSpec(shape, lambda i: (0, 0))
    return pl.pallas_call(
        functools.partial(_sa_mlp_body, S),
        grid=grid,
        in_specs=[
            pl.BlockSpec((bmm * S, C), lambda i: (i, 0)),
            pl.BlockSpec((bmm, C), lambda i: (i, 0)),
            full((1, C)), full(w1t.shape), full(w2t.shape), full(w3t.shape),
        ],
        out_specs=pl.BlockSpec((bmm, Cout), lambda i: (i, 0)),
        out_shape=jax.ShapeDtypeStruct((RQ, Cout), jnp.float32),
    )(gathered, qpad, srow, w1t, w2t, w3t)


def _tproj_body(f, qpad, fc1t, b1, wqt, wkt, wvt, qout, tout):
    x = jnp.dot(f[0], fc1t[...], preferred_element_type=jnp.float32) + b1[...]
    qout[0] = jnp.dot(x, wqt[...], preferred_element_type=jnp.float32)
    kk = jnp.dot(x, wkt[...], preferred_element_type=jnp.float32)
    vv = jnp.dot(x, wvt[...], preferred_element_type=jnp.float32)
    tout[0] = jnp.concatenate([kk, vv, qpad[0]], axis=1)


def _tproj(f, qpad, fc1t, b1, wqt, wkt, wvt, bmp):
    B, M, din = f.shape
    d = fc1t.shape[1]
    grid = (B, M // bmp)
    full = lambda shape: pl.BlockSpec(shape, lambda b, i: (0, 0))
    return pl.pallas_call(
        _tproj_body,
        grid=grid,
        in_specs=[
            pl.BlockSpec((1, bmp, din), lambda b, i: (b, i, 0)),
            pl.BlockSpec((1, bmp, 16), lambda b, i: (b, i, 0)),
            full(fc1t.shape), full(b1.shape), full(wqt.shape),
            full(wkt.shape), full(wvt.shape),
        ],
        out_specs=[
            pl.BlockSpec((1, bmp, d), lambda b, i: (b, i, 0)),
            pl.BlockSpec((1, bmp, 2 * d + 16), lambda b, i: (b, i, 0)),
        ],
        out_shape=[
            jax.ShapeDtypeStruct((B, M, d), jnp.float32),
            jax.ShapeDtypeStruct((B, M, 2 * d + 16), jnp.float32),
        ],
    )(f, qpad, fc1t, b1, wqt, wkt, wvt)


def _tknn_body(K, N, qx, qy, qz, xs, ys, zs, idx_out):
    b = pl.program_id(0)
    BM = qx.shape[1]
    qxc = jnp.reshape(qx[0, :], (BM, 1))
    qyc = jnp.reshape(qy[0, :], (BM, 1))
    qzc = jnp.reshape(qz[0, :], (BM, 1))
    xr = jnp.reshape(xs[0, :], (1, N))
    yr = jnp.reshape(ys[0, :], (1, N))
    zr = jnp.reshape(zs[0, :], (1, N))
    qsq = qxc * qxc + qyc * qyc + qzc * qzc
    xsq = xr * xr + yr * yr + zr * zr
    dot = qxc * xr + qyc * yr + qzc * zr
    D = jnp.maximum(qsq - 2.0 * dot + xsq, 0.0)
    iota = lax.broadcasted_iota(jnp.int32, (BM, N), 1)

    def step(s, D):
        m = jnp.min(D, axis=1, keepdims=True)
        sel = jnp.min(jnp.where(D == m, iota, N), axis=1, keepdims=True)
        idx_out[0, :, pl.ds(s, 1)] = (sel + b * N).astype(jnp.int32)
        return jnp.where(iota == sel, jnp.float32(np.inf), D)

    lax.fori_loop(0, K, step, D, unroll=True)


def _tknn(xyz, K, bm):
    B, M, _ = xyz.shape
    qx, qy, qz = (xyz[:, :, i] for i in range(3))
    grid = (B, M // bm)
    qspec = pl.BlockSpec((1, bm), lambda b, i: (b, i))
    xspec = pl.BlockSpec((1, M), lambda b, i: (b, 0))
    return pl.pallas_call(
        functools.partial(_tknn_body, K, M),
        grid=grid,
        in_specs=[qspec] * 3 + [xspec] * 3,
        out_specs=pl.BlockSpec((1, bm, K), lambda b, i: (b, i, 0)),
        out_shape=jax.ShapeDtypeStruct((B, M, K), jnp.int32),
    )(qx, qy, qz, qx, qy, qz)


def _tattn_body(K, d, g, q, qpad, pre, d1t, bd1, d2t, bd2, g1t, bg1, g2t,
                bg2, fc2t, bfc2, out):
    BMA = q.shape[0]
    R = BMA * K
    gg = g[...]
    kk = gg[:, :d]
    vv = gg[:, d:2 * d]
    kx = gg[:, 2 * d:]
    qxr = jnp.reshape(
        jnp.broadcast_to(qpad[...][:, None, :], (BMA, K, 16)), (R, 16))
    delta = qxr - kx
    pos = jnp.maximum(
        jnp.dot(delta, d1t[...], preferred_element_type=jnp.float32)
        + bd1[...], 0.0)
    pos = jnp.dot(pos, d2t[...], preferred_element_type=jnp.float32) + bd2[...]
    qrep = jnp.reshape(
        jnp.broadcast_to(q[...][:, None, :], (BMA, K, d)), (R, d))
    gq = qrep - kk + pos
    attn = jnp.maximum(
        jnp.dot(gq, g1t[...], preferred_element_type=jnp.float32)
        + bg1[...], 0.0)
    attn = (jnp.dot(attn, g2t[...], preferred_element_type=jnp.float32)
            + bg2[...])
    a3 = jnp.reshape(attn * jnp.float32(1.0 / np.sqrt(d)), (BMA, K, d))
    mx = jnp.max(a3, axis=1, keepdims=True)
    e = jnp.exp(a3 - mx)
    w = e / jnp.sum(e, axis=1, keepdims=True)
    contrib = w * jnp.reshape(vv + pos, (BMA, K, d))
    res = jnp.sum(contrib, axis=1)
    out[...] = (jnp.dot(res, fc2t[...], preferred_element_type=jnp.float32)
                + bfc2[...] + pre[...])


def _tattn(gathered, q2, qpad2, pre2, p, K, d, bma):
    RQ = q2.shape[0]
    d1t = jnp.pad(p['d1_w'], ((0, 0), (0, 13))).T
    args = [gathered, q2, qpad2, pre2,
            d1t, p['d1_b'][None, :], p['d2_w'].T, p['d2_b'][None, :],
            p['g1_w'].T, p['g1_b'][None, :], p['g2_w'].T, p['g2_b'][None, :],
            p['fc2_w'].T, p['fc2_b'][None, :]]
    grid = (RQ // bma,)
    full = lambda a: pl.BlockSpec(a.shape, lambda i: (0, 0))
    return pl.pallas_call(
        functools.partial(_tattn_body, K, d),
        grid=grid,
        in_specs=[
            pl.BlockSpec((bma * K, 2 * d + 16), lambda i: (i, 0)),
            pl.BlockSpec((bma, d), lambda i: (i, 0)),
            pl.BlockSpec((bma, 16), lambda i: (i, 0)),
            pl.BlockSpec((bma, d), lambda i: (i, 0)),
        ] + [full(a) for a in args[4:]],
        out_specs=pl.BlockSpec((bma, d), lambda i: (i, 0)),
        out_shape=jax.ShapeDtypeStruct((RQ, d), jnp.float32),
    )(*args)


def _sa_module(xyz, feats, npoint, radius, nsample, weights, bm, bmm,
               tbl=None):
    B, N, _ = xyz.shape
    C = feats.shape[2]
    inds, fp2, nx, ny, nz = _fps(xyz, npoint, tbl)
    new_xyz = jnp.stack([nx, ny, nz], axis=-1)
    idx = _ball_query(new_xyz, xyz, radius, nsample, bm)

    Cp = 16 if C == 3 else ((3 + C + 15) // 16) * 16
    table = jnp.concatenate(
        [xyz, feats, jnp.zeros((B, N, Cp - 3 - C), jnp.float32)],
        axis=2).reshape(B * N, Cp)
    gathered = _sc_gather(table, idx.reshape(-1, 128))

    qpad = jnp.pad(new_xyz.reshape(B * npoint, 3), ((0, 0), (0, Cp - 3)))
    srow = jnp.concatenate(
        [jnp.full((1, 3), 1.0 / radius, jnp.float32),
         jnp.ones((1, C), jnp.float32),
         jnp.zeros((1, Cp - 3 - C), jnp.float32)], axis=1)
    w1, w2, w3 = weights
    w1t = jnp.pad(w1, ((0, 0), (0, Cp - 3 - C))).T * _BN
    nf = _sa_mlp(gathered, qpad, srow, w1t, w2.T * _BN, w3.T * _BN,
                 nsample, bmm)
    return new_xyz, nf.reshape(B, npoint, w3.shape[0]), inds, fp2


def _transformer(xyz, f, p, K, bm, bmp, bma):
    B, M, d = f.shape
    qpad3 = jnp.pad(xyz.reshape(B * M, 3),
                    ((0, 0), (0, 13))).reshape(B, M, 16)
    q, tablef = _tproj(f, qpad3, p['fc1_w'].T, p['fc1_b'][None, :],
                       p['wq'].T, p['wk'].T, p['wv'].T, bmp)
    knn = _tknn(xyz, K, bm)
    gathered = _sc_gather(tablef.reshape(B * M, 2 * d + 16),
                          knn.reshape(-1, 128))
    res = _tattn(gathered, q.reshape(B * M, d), qpad3.reshape(B * M, 16),
                 f.reshape(B * M, d), p, K, d, bma)
    return res.reshape(B, M, d)


def kernel(pointcloud, params):
    xyz = pointcloud[..., :3]
    feats = pointcloud[..., 3:]
    xyz1, f1, inds1, _ = _sa_module(
        xyz, feats, 2048, 0.04, 64, params['sa1'], bm=128, bmm=64)
    f1 = _transformer(xyz1, f1, params['t1'], 16, bm=256, bmp=512, bma=128)
    xyz2, f2, inds2, fp2_inds = _sa_module(
        xyz1, f1, 1024, 0.1, 32, params['sa2'], bm=128, bmm=128, tbl=inds1)
    f2 = _transformer(xyz2, f2, params['t2'], 16, bm=256, bmp=512, bma=128)
    return jnp.transpose(f2, (0, 2, 1)), xyz2, fp2_inds


# speedup vs baseline: 10.0686x; 10.0686x over previous
import functools

import jax
import jax.numpy as jnp
import numpy as np
from jax import lax
from jax.experimental import pallas as pl
from jax.experimental.pallas import tpu as pltpu
from jax.experimental.pallas import tpu_sc as plsc

_BN = 1.0 / np.sqrt(1.0 + 1e-5)


def _b16(v):
    return v.astype(jnp.bfloat16).astype(jnp.float32)


def _fps_body(npoint, sub, lanes, has_tbl, *refs):
    if has_tbl:
        xs, ys, zs, tbl, inds, fp2, nx, ny, nz, dists = refs
    else:
        xs, ys, zs, inds, nx, ny, nz, dists = refs
        tbl = fp2 = None
    B = xs.shape[0]
    NS = npoint // 128
    iota = (lax.broadcasted_iota(jnp.int32, (sub, lanes), 0) * lanes
            + lax.broadcasted_iota(jnp.int32, (sub, lanes), 1))
    oiota = (lax.broadcasted_iota(jnp.int32, (NS, 128), 0) * 128
             + lax.broadcasted_iota(jnp.int32, (NS, 128), 1))
    NINF = jnp.float32(-np.inf)

    dists[...] = jnp.full((B, sub, lanes), 1e10, jnp.float32)

    def acc0(dt):
        return tuple(jnp.zeros((NS, 128), dt) for _ in range(B))

    def put(accs, b, i, v):
        return tuple(jnp.where(oiota == i, v, a) if bb == b else a
                     for bb, a in enumerate(accs))

    def step(i, carry):
        fs, ai, af, ax, ay, az = carry
        nfs = []
        for b in range(B):
            f = fs[b]
            sel = iota == f
            xb = xs[b]
            yb = ys[b]
            zb = zs[b]
            cx = jnp.max(jnp.where(sel, xb, NINF))
            cy = jnp.max(jnp.where(sel, yb, NINF))
            cz = jnp.max(jnp.where(sel, zb, NINF))
            ai = put(ai, b, i, f)
            ax = put(ax, b, i, cx)
            ay = put(ay, b, i, cy)
            az = put(az, b, i, cz)
            if has_tbl:
                af = put(af, b, i, jnp.max(jnp.where(sel, tbl[b], -1)))
            dx = xb - cx
            dy = yb - cy
            dz = zb - cz
            d = (dx * dx + dy * dy) + dz * dz
            dmin = jnp.minimum(dists[b], d)
            dists[b] = dmin
            m = jnp.max(dmin)
            nfs.append(jnp.min(jnp.where(dmin == m, iota, sub * lanes)))
        return tuple(nfs), ai, af, ax, ay, az

    init = (tuple(jnp.int32(0) for _ in range(B)), acc0(jnp.int32),
            acc0(jnp.int32), acc0(jnp.float32), acc0(jnp.float32),
            acc0(jnp.float32))
    _, ai, af, ax, ay, az = lax.fori_loop(0, npoint, step, init,
                                          unroll=False)
    for b in range(B):
        inds[b] = ai[b]
        nx[b] = ax[b]
        ny[b] = ay[b]
        nz[b] = az[b]
        if has_tbl:
            fp2[b] = af[b]


def _fps(xyz, npoint, tbl=None):
    B, N, _ = xyz.shape
    lanes = 1024
    sub = N // lanes
    NS = npoint // 128
    xs = xyz[:, :, 0].reshape(B, sub, lanes)
    ys = xyz[:, :, 1].reshape(B, sub, lanes)
    zs = xyz[:, :, 2].reshape(B, sub, lanes)
    out_types = [jax.ShapeDtypeStruct((B, NS, 128), jnp.int32)]
    if tbl is not None:
        out_types.append(jax.ShapeDtypeStruct((B, NS, 128), jnp.int32))
        tbl = tbl.reshape(B, N // lanes, lanes)
    out_types += [jax.ShapeDtypeStruct((B, NS, 128), jnp.float32)] * 3
    args = (xs, ys, zs) + (() if tbl is None else (tbl,))
    outs = pl.pallas_call(
        functools.partial(_fps_body, npoint, sub, lanes, tbl is not None),
        out_shape=tuple(out_types),
        scratch_shapes=[pltpu.VMEM((B, sub, lanes), jnp.float32)],
    )(*args)
    outs = tuple(o.reshape(B, npoint) for o in outs)
    if tbl is not None:
        return outs
    inds, nx, ny, nz = outs
    return inds, None, nx, ny, nz


def _bq_body(nsample, r2, N, qx, qy, qz, xs, ys, zs, idx_out, val):
    b = pl.program_id(0)
    BM = qx.shape[-2]
    qxc = qx[0, 0]
    qyc = qy[0, 0]
    qzc = qz[0, 0]
    qsq = qxc * qxc + qyc * qyc + qzc * qzc
    xr = xs[0]
    yr = ys[0]
    zr = zs[0]
    xsq = xr * xr + yr * yr + zr * zr
    dot = (_b16(qxc) * _b16(xr) + _b16(qyc) * _b16(yr)) \
        + _b16(qzc) * _b16(zr)
    D = jnp.maximum((qsq - 2.0 * dot) + xsq, 0.0)
    io = lax.broadcasted_iota(jnp.int32, (BM, N), 1)
    val[...] = jnp.where(D < r2, io, N)

    v = val[...]
    m0 = jnp.min(v, axis=1, keepdims=True)
    val[...] = jnp.where(v == m0, N, v)
    out0 = jnp.broadcast_to(m0, (BM, nsample))

    def cond(carry):
        s, _, active = carry
        return jnp.logical_and(s < nsample, active)

    def body(carry):
        s, out_acc, _ = carry
        v = val[...]
        m = jnp.min(v, axis=1, keepdims=True)
        live = m < N
        rec = jnp.where(live, m, out_acc[:, 0:1])
        kio = lax.broadcasted_iota(jnp.int32, (BM, nsample), 1)
        out_acc = jnp.where(kio == s, jnp.broadcast_to(rec, (BM, nsample)),
                            out_acc)
        val[...] = jnp.where(v == m, N, v)
        return s + 1, out_acc, jnp.any(live)

    _, out_acc, _ = lax.while_loop(
        cond, body, (jnp.int32(1), out0, jnp.bool_(True)))
    idx_out[0] = jnp.minimum(out_acc, N - 1) + b * N


def _ball_query(new_xyz, xyz, radius, nsample, bm):
    B, M, _ = new_xyz.shape
    N = xyz.shape[1]
    qx, qy, qz = (new_xyz[:, :, i].reshape(B, M // bm, bm, 1)
                  for i in range(3))
    xs, ys, zs = (xyz[:, :, i].reshape(B, 1, N) for i in range(3))
    grid = (B, M // bm)
    qspec = pl.BlockSpec((1, 1, bm, 1), lambda b, i: (b, i, 0, 0))
    xspec = pl.BlockSpec((1, 1, N), lambda b, i: (b, 0, 0))
    return pl.pallas_call(
        functools.partial(_bq_body, nsample, radius * radius, N),
        grid=grid,
        in_specs=[qspec] * 3 + [xspec] * 3,
        out_specs=pl.BlockSpec((1, bm, nsample), lambda b, i: (b, i, 0)),
        out_shape=jax.ShapeDtypeStruct((B, M, nsample), jnp.int32),
        scratch_shapes=[pltpu.VMEM((bm, N), jnp.int32)],
    )(qx, qy, qz, xs, ys, zs)


def _tknn_body(K, N, qx, qy, qz, xs, ys, zs, idx_out, val):
    b = pl.program_id(0)
    BM = qx.shape[-2]
    INF = jnp.float32(np.inf)
    qxc = qx[0, 0]
    qyc = qy[0, 0]
    qzc = qz[0, 0]
    qsq = qxc * qxc + qyc * qyc + qzc * qzc
    xr = xs[0]
    yr = ys[0]
    zr = zs[0]
    xsq = xr * xr + yr * yr + zr * zr
    dot = (_b16(qxc) * _b16(xr) + _b16(qyc) * _b16(yr)) \
        + _b16(qzc) * _b16(zr)
    val[...] = jnp.maximum((qsq - 2.0 * dot) + xsq, 0.0)
    io = lax.broadcasted_iota(jnp.int32, (BM, N), 1)

    def step(s, out_acc):
        v = val[...]
        m = jnp.min(v, axis=1, keepdims=True)
        sel = jnp.min(jnp.where(v == m, io, N), axis=1, keepdims=True)
        val[...] = jnp.where(io == sel, INF, v)
        kio = lax.broadcasted_iota(jnp.int32, (BM, K), 1)
        return jnp.where(kio == s, jnp.broadcast_to(sel, (BM, K)), out_acc)

    out_acc = lax.fori_loop(0, K, step, jnp.zeros((BM, K), jnp.int32),
                            unroll=False)
    idx_out[0] = out_acc + b * N


def _tknn(xyz, K, bm):
    B, M, _ = xyz.shape
    qx, qy, qz = (xyz[:, :, i].reshape(B, M // bm, bm, 1) for i in range(3))
    xs, ys, zs = (xyz[:, :, i].reshape(B, 1, M) for i in range(3))
    grid = (B, M // bm)
    qspec = pl.BlockSpec((1, 1, bm, 1), lambda b, i: (b, i, 0, 0))
    xspec = pl.BlockSpec((1, 1, M), lambda b, i: (b, 0, 0))
    return pl.pallas_call(
        functools.partial(_tknn_body, K, M),
        grid=grid,
        in_specs=[qspec] * 3 + [xspec] * 3,
        out_specs=pl.BlockSpec((1, bm, K), lambda b, i: (b, i, 0)),
        out_shape=jax.ShapeDtypeStruct((B, M, K), jnp.int32),
        scratch_shapes=[pltpu.VMEM((bm, M), jnp.float32)],
    )(qx, qy, qz, xs, ys, zs)


def _sc_gather(table, idx):
    D = table.shape[1]
    R = idx.shape[0]
    NW = 32
    CR = 128 if D <= 300 else 64
    idx2d = idx.reshape(R // CR, CR)
    chunks_w = R // NW // CR
    mesh = plsc.VectorSubcoreMesh(core_axis_name="c", subcore_axis_name="s")

    @functools.partial(
        pl.kernel, mesh=mesh,
        compiler_params=pltpu.CompilerParams(use_tc_tiling_on_sc=False),
        out_type=jax.ShapeDtypeStruct((R, D), jnp.float32),
        scratch_types=[
            pltpu.VMEM((chunks_w, CR), jnp.int32),
            pltpu.VMEM((CR, D), jnp.float32),
            pltpu.VMEM((CR, D), jnp.float32),
            pltpu.SemaphoreType.DMA,
            pltpu.SemaphoreType.DMA,
        ],
    )
    def k(table_hbm, idx_hbm, out_hbm, idx_v, rows0, rows1, sem0, sem1):
        wid = lax.axis_index("s") * 2 + lax.axis_index("c")
        cbase = wid * chunks_w
        pltpu.sync_copy(idx_hbm.at[pl.ds(cbase, chunks_w)], idx_v)
        rows = (rows0, rows1)
        sems = (sem0, sem1)
        pltpu.async_copy(table_hbm.at[idx_v.at[0]], rows0, sem0)

        def body(c, _):
            cur = lax.rem(c, 2)
            for par in range(2):
                @pl.when(cur == par)
                def _():
                    @pl.when(c + 1 < chunks_w)
                    def _():
                        pltpu.async_copy(
                            table_hbm.at[idx_v.at[c + 1]], rows[1 - par],
                            sems[1 - par])

                    pltpu.make_async_copy(
                        table_hbm.at[idx_v.at[c]], rows[par],
                        sems[par]).wait()
                    pltpu.sync_copy(
                        rows[par], out_hbm.at[pl.ds((cbase + c) * CR, CR)])
            return 0

        lax.fori_loop(0, chunks_w, body, 0, unroll=False)

    return k(table, idx2d)


def _sa_mlp_body(g, q, srow, w1, w2, w3, out):
    s = pl.program_id(1)
    x = (g[0] - q[...]) * srow[...]
    x = jnp.maximum(jnp.dot(_b16(x), w1[...],
                            preferred_element_type=jnp.float32) * _BN, 0.0)
    x = jnp.maximum(jnp.dot(_b16(x), w2[...],
                            preferred_element_type=jnp.float32) * _BN, 0.0)
    x = jnp.maximum(jnp.dot(_b16(x), w3[...],
                            preferred_element_type=jnp.float32) * _BN, 0.0)

    @pl.when(s == 0)
    def _():
        out[...] = x

    @pl.when(s > 0)
    def _():
        out[...] = jnp.maximum(out[...], x)


def _sa_mlp(gathered, qpad, srow, w1t, w2t, w3t, S, bmm):
    RQ, C = qpad.shape
    Cout = w3t.shape[1]
    grid = (RQ // bmm, S)
    full = lambda shape: pl.BlockSpec(shape, lambda i, s: tuple(
        0 for _ in shape))
    return pl.pallas_call(
        _sa_mlp_body,
        grid=grid,
        in_specs=[
            pl.BlockSpec((1, bmm, C), lambda i, s: (s, i, 0)),
            pl.BlockSpec((bmm, C), lambda i, s: (i, 0)),
            full((1, C)), full(w1t.shape), full(w2t.shape), full(w3t.shape),
        ],
        out_specs=pl.BlockSpec((bmm, Cout), lambda i, s: (i, 0)),
        out_shape=jax.ShapeDtypeStruct((RQ, Cout), jnp.float32),
    )(gathered, qpad, srow, w1t, w2t, w3t)


def _tproj_body(f, qpad, fc1t, b1, wqt, wkt, wvt, qout, tout):
    x = jnp.dot(_b16(f[0]), fc1t[...],
                preferred_element_type=jnp.float32) + b1[...]
    xb = _b16(x)
    qout[0] = jnp.dot(xb, wqt[...], preferred_element_type=jnp.float32)
    kk = jnp.dot(xb, wkt[...], preferred_element_type=jnp.float32)
    vv = jnp.dot(xb, wvt[...], preferred_element_type=jnp.float32)
    tout[0] = jnp.concatenate([kk, vv, qpad[0]], axis=1)


def _tproj(f, qpad, fc1t, b1, wqt, wkt, wvt, bmp):
    B, M, din = f.shape
    d = fc1t.shape[1]
    grid = (B, M // bmp)
    full = lambda shape: pl.BlockSpec(shape, lambda b, i: (0, 0))
    return pl.pallas_call(
        _tproj_body,
        grid=grid,
        in_specs=[
            pl.BlockSpec((1, bmp, din), lambda b, i: (b, i, 0)),
            pl.BlockSpec((1, bmp, 16), lambda b, i: (b, i, 0)),
            full(fc1t.shape), full(b1.shape), full(wqt.shape),
            full(wkt.shape), full(wvt.shape),
        ],
        out_specs=[
            pl.BlockSpec((1, bmp, d), lambda b, i: (b, i, 0)),
            pl.BlockSpec((1, bmp, 2 * d + 16), lambda b, i: (b, i, 0)),
        ],
        out_shape=[
            jax.ShapeDtypeStruct((B, M, d), jnp.float32),
            jax.ShapeDtypeStruct((B, M, 2 * d + 16), jnp.float32),
        ],
    )(f, qpad, fc1t, b1, wqt, wkt, wvt)


def _tattn_body(K, d, g, q, qpad, pre, d1t, bd1, d2t, bd2, g1t, bg1, g2t,
                bg2, fc2t, bfc2, out):
    bma = q.shape[0]
    R = bma * K
    gg = g[...]
    kk = gg[:, :d]
    vv = gg[:, d:2 * d]
    kx = gg[:, 2 * d:]
    qxr = jnp.reshape(
        jnp.broadcast_to(qpad[...][:, None, :], (bma, K, 16)), (R, 16))
    delta = qxr - kx
    pos = jnp.maximum(
        jnp.dot(_b16(delta), d1t[...], preferred_element_type=jnp.float32)
        + bd1[...], 0.0)
    pos = jnp.dot(_b16(pos), d2t[...],
                  preferred_element_type=jnp.float32) + bd2[...]
    qrep = jnp.reshape(
        jnp.broadcast_to(q[...][:, None, :], (bma, K, d)), (R, d))
    gq = qrep - kk + pos
    attn = jnp.maximum(
        jnp.dot(_b16(gq), g1t[...], preferred_element_type=jnp.float32)
        + bg1[...], 0.0)
    attn = (jnp.dot(_b16(attn), g2t[...], preferred_element_type=jnp.float32)
            + bg2[...])
    a3 = jnp.reshape(attn * jnp.float32(1.0 / np.sqrt(d)), (bma, K, d))
    mx = jnp.max(a3, axis=1, keepdims=True)
    e = jnp.exp(a3 - mx)
    w = e / jnp.sum(e, axis=1, keepdims=True)
    contrib = w * jnp.reshape(vv + pos, (bma, K, d))
    res = jnp.sum(contrib, axis=1)
    out[...] = (jnp.dot(_b16(res), fc2t[...],
                        preferred_element_type=jnp.float32)
                + bfc2[...] + pre[...])


def _tattn(gathered, q2, qpad2, pre2, p, K, d, bma):
    RQ = q2.shape[0]
    bt = lambda w: w.T.astype(jnp.bfloat16)
    d1t = jnp.pad(p['d1_w'], ((0, 0), (0, 13))).T.astype(jnp.bfloat16)
    args = [gathered, q2, qpad2, pre2,
            d1t, p['d1_b'][None, :], bt(p['d2_w']), p['d2_b'][None, :],
            bt(p['g1_w']), p['g1_b'][None, :], bt(p['g2_w']),
            p['g2_b'][None, :], bt(p['fc2_w']), p['fc2_b'][None, :]]
    grid = (RQ // bma,)
    full = lambda a: pl.BlockSpec(a.shape, lambda i: (0, 0))
    return pl.pallas_call(
        functools.partial(_tattn_body, K, d),
        grid=grid,
        in_specs=[
            pl.BlockSpec((bma * K, 2 * d + 16), lambda i: (i, 0)),
            pl.BlockSpec((bma, d), lambda i: (i, 0)),
            pl.BlockSpec((bma, 16), lambda i: (i, 0)),
            pl.BlockSpec((bma, d), lambda i: (i, 0)),
        ] + [full(a) for a in args[4:]],
        out_specs=pl.BlockSpec((bma, d), lambda i: (i, 0)),
        out_shape=jax.ShapeDtypeStruct((RQ, d), jnp.float32),
    )(*args)


def _sa_module(xyz, feats, npoint, radius, nsample, weights, bm, bmm,
               tbl=None):
    B, N, _ = xyz.shape
    C = feats.shape[2]
    inds, fp2, nx, ny, nz = _fps(xyz, npoint, tbl)
    new_xyz = jnp.stack([nx, ny, nz], axis=-1)
    idx = _ball_query(new_xyz, xyz, radius, nsample, bm)

    Cp = 16 if C == 3 else ((3 + C + 15) // 16) * 16
    table = jnp.concatenate(
        [xyz, feats, jnp.zeros((B, N, Cp - 3 - C), jnp.float32)],
        axis=2).reshape(B * N, Cp)
    idx_sm = jnp.transpose(idx, (2, 0, 1)).reshape(-1)
    gathered = _sc_gather(table, idx_sm).reshape(nsample, B * npoint, Cp)

    qpad = jnp.pad(new_xyz.reshape(B * npoint, 3), ((0, 0), (0, Cp - 3)))
    srow = jnp.concatenate(
        [jnp.full((1, 3), 1.0 / radius, jnp.float32),
         jnp.ones((1, C), jnp.float32),
         jnp.zeros((1, Cp - 3 - C), jnp.float32)], axis=1)
    w1, w2, w3 = weights
    w1t = jnp.pad(w1, ((0, 0), (0, Cp - 3 - C))).T.astype(jnp.bfloat16)
    nf = _sa_mlp(gathered, qpad, srow, w1t, w2.T.astype(jnp.bfloat16),
                 w3.T.astype(jnp.bfloat16), nsample, bmm)
    return new_xyz, nf.reshape(B, npoint, w3.shape[0]), inds, fp2


def _transformer(xyz, f, p, K, bm, bmp, bma):
    B, M, d = f.shape
    qpad3 = jnp.pad(xyz.reshape(B * M, 3),
                    ((0, 0), (0, 13))).reshape(B, M, 16)
    q, tablef = _tproj(f, qpad3, p['fc1_w'].T.astype(jnp.bfloat16),
                       p['fc1_b'][None, :],
                       p['wq'].T.astype(jnp.bfloat16),
                       p['wk'].T.astype(jnp.bfloat16),
                       p['wv'].T.astype(jnp.bfloat16), bmp)
    knn = _tknn(xyz, K, bm)
    knn_qm = knn.reshape(-1)
    gathered = _sc_gather(tablef.reshape(B * M, 2 * d + 16), knn_qm)
    res = _tattn(gathered, q.reshape(B * M, d), qpad3.reshape(B * M, 16),
                 f.reshape(B * M, d), p, K, d, bma)
    return res.reshape(B, M, d)


def kernel(pointcloud, params):
    xyz = pointcloud[..., :3]
    feats = pointcloud[..., 3:]
    xyz1, f1, inds1, _ = _sa_module(
        xyz, feats, 2048, 0.04, 64, params['sa1'], bm=64, bmm=512)
    f1 = _transformer(xyz1, f1, params['t1'], 16, bm=128, bmp=256, bma=32)
    xyz2, f2, inds2, fp2_inds = _sa_module(
        xyz1, f1, 1024, 0.1, 32, params['sa2'], bm=128, bmm=256, tbl=inds1)
    f2 = _transformer(xyz2, f2, params['t2'], 16, bm=128, bmp=256, bma=16)
    return jnp.transpose(f2, (0, 2, 1)), xyz2, fp2_inds


# speedup vs baseline: 15.1887x; 1.5085x over previous
import functools

import jax
import jax.numpy as jnp
import numpy as np
from jax import lax
from jax.experimental import pallas as pl
from jax.experimental.pallas import tpu as pltpu
from jax.experimental.pallas import tpu_sc as plsc

_BN = 1.0 / np.sqrt(1.0 + 1e-5)


def _b16(v):
    return v.astype(jnp.bfloat16).astype(jnp.float32)


def _fps_body(npoint, sub, lanes, has_tbl, *refs):
    if has_tbl:
        xs, ys, zs, tbl, inds, fp2, nx, ny, nz, dists = refs
    else:
        xs, ys, zs, inds, nx, ny, nz, dists = refs
        tbl = fp2 = None
    B = xs.shape[0]
    NS = npoint // 128
    iota3 = jnp.broadcast_to(
        (lax.broadcasted_iota(jnp.int32, (sub, lanes), 0) * lanes
         + lax.broadcasted_iota(jnp.int32, (sub, lanes), 1))[None],
        (B, sub, lanes))
    oiota = jnp.broadcast_to(
        (lax.broadcasted_iota(jnp.int32, (NS, 128), 0) * 128
         + lax.broadcasted_iota(jnp.int32, (NS, 128), 1))[None],
        (B, NS, 128))
    NINF = jnp.float32(-np.inf)

    dists[...] = jnp.full((B, sub, lanes), 1e10, jnp.float32)

    def ext(sel, arr, fill):
        return jnp.max(jnp.where(sel, arr, fill), axis=(1, 2),
                       keepdims=True)

    def step(i, carry):
        f, ai, af, ax, ay, az = carry
        sel = iota3 == f
        xb = xs[...]
        yb = ys[...]
        zb = zs[...]
        cx = ext(sel, xb, NINF)
        cy = ext(sel, yb, NINF)
        cz = ext(sel, zb, NINF)
        here = oiota == i
        ai = jnp.where(here, f, ai)
        ax = jnp.where(here, cx, ax)
        ay = jnp.where(here, cy, ay)
        az = jnp.where(here, cz, az)
        if has_tbl:
            af = jnp.where(here, ext(sel, tbl[...], -1), af)
        dx = xb - cx
        dy = yb - cy
        dz = zb - cz
        d = (dx * dx + dy * dy) + dz * dz
        dmin = jnp.minimum(dists[...], d)
        dists[...] = dmin
        m = jnp.max(dmin, axis=(1, 2), keepdims=True)
        nf = jnp.min(jnp.where(dmin == m, iota3, sub * lanes), axis=(1, 2),
                     keepdims=True)
        return nf, ai, af, ax, ay, az

    z = lambda dt: jnp.zeros((B, NS, 128), dt)
    _, ai, af, ax, ay, az = lax.fori_loop(
        0, npoint, step,
        (jnp.zeros((B, 1, 1), jnp.int32), z(jnp.int32), z(jnp.int32),
         z(jnp.float32), z(jnp.float32), z(jnp.float32)),
        unroll=False)
    inds[...] = ai
    nx[...] = ax
    ny[...] = ay
    nz[...] = az
    if has_tbl:
        fp2[...] = af


def _fps(xyz, npoint, tbl=None):
    B, N, _ = xyz.shape
    lanes = 1024
    sub = N // lanes
    NS = npoint // 128
    xs = xyz[:, :, 0].reshape(B, sub, lanes)
    ys = xyz[:, :, 1].reshape(B, sub, lanes)
    zs = xyz[:, :, 2].reshape(B, sub, lanes)
    out_types = [jax.ShapeDtypeStruct((B, NS, 128), jnp.int32)]
    if tbl is not None:
        out_types.append(jax.ShapeDtypeStruct((B, NS, 128), jnp.int32))
        tbl = tbl.reshape(B, N // lanes, lanes)
    out_types += [jax.ShapeDtypeStruct((B, NS, 128), jnp.float32)] * 3
    args = (xs, ys, zs) + (() if tbl is None else (tbl,))
    outs = pl.pallas_call(
        functools.partial(_fps_body, npoint, sub, lanes, tbl is not None),
        out_shape=tuple(out_types),
        scratch_shapes=[pltpu.VMEM((B, sub, lanes), jnp.float32)],
    )(*args)
    outs = tuple(o.reshape(B, npoint) for o in outs)
    if tbl is not None:
        return outs
    inds, nx, ny, nz = outs
    return inds, None, nx, ny, nz


def _bq_body(nsample, r2, N, qx, qy, qz, xs, ys, zs, idx_out, val):
    b = pl.program_id(0)
    BM = qx.shape[-2]
    qxc = qx[0, 0]
    qyc = qy[0, 0]
    qzc = qz[0, 0]
    qsq = qxc * qxc + qyc * qyc + qzc * qzc
    xr = xs[0]
    yr = ys[0]
    zr = zs[0]
    xsq = xr * xr + yr * yr + zr * zr
    dot = (_b16(qxc) * _b16(xr) + _b16(qyc) * _b16(yr)) \
        + _b16(qzc) * _b16(zr)
    D = jnp.maximum((qsq - 2.0 * dot) + xsq, 0.0)
    io = lax.broadcasted_iota(jnp.int32, (BM, N), 1)
    val[...] = jnp.where(D < r2, io, N)

    v = val[...]
    m0 = jnp.min(v, axis=1, keepdims=True)
    val[...] = jnp.where(v == m0, N, v)
    out0 = jnp.broadcast_to(m0, (BM, nsample))

    def cond(carry):
        s, _, active = carry
        return jnp.logical_and(s < nsample, active)

    def body(carry):
        s, out_acc, _ = carry
        v = val[...]
        m = jnp.min(v, axis=1, keepdims=True)
        live = m < N
        rec = jnp.where(live, m, out_acc[:, 0:1])
        kio = lax.broadcasted_iota(jnp.int32, (BM, nsample), 1)
        out_acc = jnp.where(kio == s, jnp.broadcast_to(rec, (BM, nsample)),
                            out_acc)
        val[...] = jnp.where(v == m, N, v)
        return s + 1, out_acc, jnp.any(live)

    _, out_acc, _ = lax.while_loop(
        cond, body, (jnp.int32(1), out0, jnp.bool_(True)))
    idx_out[0] = jnp.minimum(out_acc, N - 1) + b * N


def _ball_query(new_xyz, xyz, radius, nsample, bm):
    B, M, _ = new_xyz.shape
    N = xyz.shape[1]
    qx, qy, qz = (new_xyz[:, :, i].reshape(B, M // bm, bm, 1)
                  for i in range(3))
    xs, ys, zs = (xyz[:, :, i].reshape(B, 1, N) for i in range(3))
    grid = (B, M // bm)
    qspec = pl.BlockSpec((1, 1, bm, 1), lambda b, i: (b, i, 0, 0))
    xspec = pl.BlockSpec((1, 1, N), lambda b, i: (b, 0, 0))
    return pl.pallas_call(
        functools.partial(_bq_body, nsample, radius * radius, N),
        grid=grid,
        in_specs=[qspec] * 3 + [xspec] * 3,
        out_specs=pl.BlockSpec((1, bm, nsample), lambda b, i: (b, i, 0)),
        out_shape=jax.ShapeDtypeStruct((B, M, nsample), jnp.int32),
        scratch_shapes=[pltpu.VMEM((bm, N), jnp.int32)],
    )(qx, qy, qz, xs, ys, zs)


def _tknn_body(K, N, qx, qy, qz, xs, ys, zs, idx_out, val):
    b = pl.program_id(0)
    BM = qx.shape[-2]
    INF = jnp.float32(np.inf)
    qxc = qx[0, 0]
    qyc = qy[0, 0]
    qzc = qz[0, 0]
    qsq = qxc * qxc + qyc * qyc + qzc * qzc
    xr = xs[0]
    yr = ys[0]
    zr = zs[0]
    xsq = xr * xr + yr * yr + zr * zr
    dot = (_b16(qxc) * _b16(xr) + _b16(qyc) * _b16(yr)) \
        + _b16(qzc) * _b16(zr)
    val[...] = jnp.maximum((qsq - 2.0 * dot) + xsq, 0.0)
    io = lax.broadcasted_iota(jnp.int32, (BM, N), 1)

    def step(s, out_acc):
        v = val[...]
        m = jnp.min(v, axis=1, keepdims=True)
        sel = jnp.min(jnp.where(v == m, io, N), axis=1, keepdims=True)
        val[...] = jnp.where(io == sel, INF, v)
        kio = lax.broadcasted_iota(jnp.int32, (BM, K), 1)
        return jnp.where(kio == s, jnp.broadcast_to(sel, (BM, K)), out_acc)

    out_acc = lax.fori_loop(0, K, step, jnp.zeros((BM, K), jnp.int32),
                            unroll=False)
    idx_out[0] = out_acc + b * N


def _tknn(xyz, K, bm):
    B, M, _ = xyz.shape
    qx, qy, qz = (xyz[:, :, i].reshape(B, M // bm, bm, 1) for i in range(3))
    xs, ys, zs = (xyz[:, :, i].reshape(B, 1, M) for i in range(3))
    grid = (B, M // bm)
    qspec = pl.BlockSpec((1, 1, bm, 1), lambda b, i: (b, i, 0, 0))
    xspec = pl.BlockSpec((1, 1, M), lambda b, i: (b, 0, 0))
    return pl.pallas_call(
        functools.partial(_tknn_body, K, M),
        grid=grid,
        in_specs=[qspec] * 3 + [xspec] * 3,
        out_specs=pl.BlockSpec((1, bm, K), lambda b, i: (b, i, 0)),
        out_shape=jax.ShapeDtypeStruct((B, M, K), jnp.int32),
        scratch_shapes=[pltpu.VMEM((bm, M), jnp.float32)],
    )(qx, qy, qz, xs, ys, zs)


def _sc_gather(table, idx):
    D = table.shape[1]
    R = idx.shape[0]
    NW = 32
    CR = 128 if D <= 300 else 64
    idx2d = idx.reshape(R // CR, CR)
    chunks_w = R // NW // CR
    mesh = plsc.VectorSubcoreMesh(core_axis_name="c", subcore_axis_name="s")

    @functools.partial(
        pl.kernel, mesh=mesh,
        compiler_params=pltpu.CompilerParams(use_tc_tiling_on_sc=False),
        out_type=jax.ShapeDtypeStruct((R, D), jnp.float32),
        scratch_types=[
            pltpu.VMEM((chunks_w, CR), jnp.int32),
            pltpu.VMEM((CR, D), jnp.float32),
            pltpu.VMEM((CR, D), jnp.float32),
            pltpu.SemaphoreType.DMA,
            pltpu.SemaphoreType.DMA,
        ],
    )
    def k(table_hbm, idx_hbm, out_hbm, idx_v, rows0, rows1, sem0, sem1):
        wid = lax.axis_index("s") * 2 + lax.axis_index("c")
        cbase = wid * chunks_w
        pltpu.sync_copy(idx_hbm.at[pl.ds(cbase, chunks_w)], idx_v)
        rows = (rows0, rows1)
        sems = (sem0, sem1)
        pltpu.async_copy(table_hbm.at[idx_v.at[0]], rows0, sem0)

        def body(c, _):
            cur = lax.rem(c, 2)
            for par in range(2):
                @pl.when(cur == par)
                def _():
                    @pl.when(c + 1 < chunks_w)
                    def _():
                        pltpu.async_copy(
                            table_hbm.at[idx_v.at[c + 1]], rows[1 - par],
                            sems[1 - par])

                    pltpu.make_async_copy(
                        table_hbm.at[idx_v.at[c]], rows[par],
                        sems[par]).wait()
                    pltpu.sync_copy(
                        rows[par], out_hbm.at[pl.ds((cbase + c) * CR, CR)])
            return 0

        lax.fori_loop(0, chunks_w, body, 0, unroll=False)

    return k(table, idx2d)


def _sa_mlp_body(g, q, srow, w1, w2, w3, out):
    s = pl.program_id(1)
    x = (g[0] - q[...]) * srow[...]
    x = jnp.maximum(jnp.dot(_b16(x), w1[...],
                            preferred_element_type=jnp.float32) * _BN, 0.0)
    x = jnp.maximum(jnp.dot(_b16(x), w2[...],
                            preferred_element_type=jnp.float32) * _BN, 0.0)
    x = jnp.maximum(jnp.dot(_b16(x), w3[...],
                            preferred_element_type=jnp.float32) * _BN, 0.0)

    @pl.when(s == 0)
    def _():
        out[...] = x

    @pl.when(s > 0)
    def _():
        out[...] = jnp.maximum(out[...], x)


def _sa_mlp(gathered, qpad, srow, w1t, w2t, w3t, S, bmm):
    RQ, C = qpad.shape
    Cout = w3t.shape[1]
    grid = (RQ // bmm, S)
    full = lambda shape: pl.BlockSpec(shape, lambda i, s: tuple(
        0 for _ in shape))
    return pl.pallas_call(
        _sa_mlp_body,
        grid=grid,
        in_specs=[
            pl.BlockSpec((1, bmm, C), lambda i, s: (s, i, 0)),
            pl.BlockSpec((bmm, C), lambda i, s: (i, 0)),
            full((1, C)), full(w1t.shape), full(w2t.shape), full(w3t.shape),
        ],
        out_specs=pl.BlockSpec((bmm, Cout), lambda i, s: (i, 0)),
        out_shape=jax.ShapeDtypeStruct((RQ, Cout), jnp.float32),
    )(gathered, qpad, srow, w1t, w2t, w3t)


def _tproj_body(f, qpad, fc1t, b1, wqt, wkt, wvt, qout, tout):
    x = jnp.dot(_b16(f[0]), fc1t[...],
                preferred_element_type=jnp.float32) + b1[...]
    xb = _b16(x)
    qout[0] = jnp.dot(xb, wqt[...], preferred_element_type=jnp.float32)
    kk = jnp.dot(xb, wkt[...], preferred_element_type=jnp.float32)
    vv = jnp.dot(xb, wvt[...], preferred_element_type=jnp.float32)
    tout[0] = jnp.concatenate([kk, vv, qpad[0]], axis=1)


def _tproj(f, qpad, fc1t, b1, wqt, wkt, wvt, bmp):
    B, M, din = f.shape
    d = fc1t.shape[1]
    grid = (B, M // bmp)
    full = lambda shape: pl.BlockSpec(shape, lambda b, i: (0, 0))
    return pl.pallas_call(
        _tproj_body,
        grid=grid,
        in_specs=[
            pl.BlockSpec((1, bmp, din), lambda b, i: (b, i, 0)),
            pl.BlockSpec((1, bmp, 16), lambda b, i: (b, i, 0)),
            full(fc1t.shape), full(b1.shape), full(wqt.shape),
            full(wkt.shape), full(wvt.shape),
        ],
        out_specs=[
            pl.BlockSpec((1, bmp, d), lambda b, i: (b, i, 0)),
            pl.BlockSpec((1, bmp, 2 * d + 16), lambda b, i: (b, i, 0)),
        ],
        out_shape=[
            jax.ShapeDtypeStruct((B, M, d), jnp.float32),
            jax.ShapeDtypeStruct((B, M, 2 * d + 16), jnp.float32),
        ],
    )(f, qpad, fc1t, b1, wqt, wkt, wvt)


def _tattn_body(K, d, g, q, qpad, pre, d1t, bd1, d2t, bd2, g1t, bg1, g2t,
                bg2, fc2t, bfc2, out):
    bma = q.shape[0]
    R = bma * K
    gg = g[...]
    kk = gg[:, :d]
    vv = gg[:, d:2 * d]
    kx = gg[:, 2 * d:]
    qxr = jnp.reshape(
        jnp.broadcast_to(qpad[...][:, None, :], (bma, K, 16)), (R, 16))
    delta = qxr - kx
    pos = jnp.maximum(
        jnp.dot(_b16(delta), d1t[...], preferred_element_type=jnp.float32)
        + bd1[...], 0.0)
    pos = jnp.dot(_b16(pos), d2t[...],
                  preferred_element_type=jnp.float32) + bd2[...]
    qrep = jnp.reshape(
        jnp.broadcast_to(q[...][:, None, :], (bma, K, d)), (R, d))
    gq = qrep - kk + pos
    attn = jnp.maximum(
        jnp.dot(_b16(gq), g1t[...], preferred_element_type=jnp.float32)
        + bg1[...], 0.0)
    attn = (jnp.dot(_b16(attn), g2t[...], preferred_element_type=jnp.float32)
            + bg2[...])
    a3 = jnp.reshape(attn * jnp.float32(1.0 / np.sqrt(d)), (bma, K, d))
    mx = jnp.max(a3, axis=1, keepdims=True)
    e = jnp.exp(a3 - mx)
    w = e / jnp.sum(e, axis=1, keepdims=True)
    contrib = w * jnp.reshape(vv + pos, (bma, K, d))
    res = jnp.sum(contrib, axis=1)
    out[...] = (jnp.dot(_b16(res), fc2t[...],
                        preferred_element_type=jnp.float32)
                + bfc2[...] + pre[...])


def _tattn(gathered, q2, qpad2, pre2, p, K, d, bma):
    RQ = q2.shape[0]
    bt = lambda w: w.T.astype(jnp.bfloat16)
    d1t = jnp.pad(p['d1_w'], ((0, 0), (0, 13))).T.astype(jnp.bfloat16)
    args = [gathered, q2, qpad2, pre2,
            d1t, p['d1_b'][None, :], bt(p['d2_w']), p['d2_b'][None, :],
            bt(p['g1_w']), p['g1_b'][None, :], bt(p['g2_w']),
            p['g2_b'][None, :], bt(p['fc2_w']), p['fc2_b'][None, :]]
    grid = (RQ // bma,)
    full = lambda a: pl.BlockSpec(a.shape, lambda i: (0, 0))
    return pl.pallas_call(
        functools.partial(_tattn_body, K, d),
        grid=grid,
        in_specs=[
            pl.BlockSpec((bma * K, 2 * d + 16), lambda i: (i, 0)),
            pl.BlockSpec((bma, d), lambda i: (i, 0)),
            pl.BlockSpec((bma, 16), lambda i: (i, 0)),
            pl.BlockSpec((bma, d), lambda i: (i, 0)),
        ] + [full(a) for a in args[4:]],
        out_specs=pl.BlockSpec((bma, d), lambda i: (i, 0)),
        out_shape=jax.ShapeDtypeStruct((RQ, d), jnp.float32),
    )(*args)


def _sa_module(xyz, feats, npoint, radius, nsample, weights, bm, bmm,
               tbl=None):
    B, N, _ = xyz.shape
    C = feats.shape[2]
    inds, fp2, nx, ny, nz = _fps(xyz, npoint, tbl)
    new_xyz = jnp.stack([nx, ny, nz], axis=-1)
    idx = _ball_query(new_xyz, xyz, radius, nsample, bm)

    Cp = 16 if C == 3 else ((3 + C + 15) // 16) * 16
    table = jnp.concatenate(
        [xyz, feats, jnp.zeros((B, N, Cp - 3 - C), jnp.float32)],
        axis=2).reshape(B * N, Cp)
    idx_sm = jnp.transpose(idx, (2, 0, 1)).reshape(-1)
    gathered = _sc_gather(table, idx_sm).reshape(nsample, B * npoint, Cp)

    qpad = jnp.pad(new_xyz.reshape(B * npoint, 3), ((0, 0), (0, Cp - 3)))
    srow = jnp.concatenate(
        [jnp.full((1, 3), 1.0 / radius, jnp.float32),
         jnp.ones((1, C), jnp.float32),
         jnp.zeros((1, Cp - 3 - C), jnp.float32)], axis=1)
    w1, w2, w3 = weights
    w1t = jnp.pad(w1, ((0, 0), (0, Cp - 3 - C))).T.astype(jnp.bfloat16)
    nf = _sa_mlp(gathered, qpad, srow, w1t, w2.T.astype(jnp.bfloat16),
                 w3.T.astype(jnp.bfloat16), nsample, bmm)
    return new_xyz, nf.reshape(B, npoint, w3.shape[0]), inds, fp2


def _transformer(xyz, f, p, K, bm, bmp, bma):
    B, M, d = f.shape
    qpad3 = jnp.pad(xyz.reshape(B * M, 3),
                    ((0, 0), (0, 13))).reshape(B, M, 16)
    q, tablef = _tproj(f, qpad3, p['fc1_w'].T.astype(jnp.bfloat16),
                       p['fc1_b'][None, :],
                       p['wq'].T.astype(jnp.bfloat16),
                       p['wk'].T.astype(jnp.bfloat16),
                       p['wv'].T.astype(jnp.bfloat16), bmp)
    knn = _tknn(xyz, K, bm)
    knn_qm = knn.reshape(-1)
    gathered = _sc_gather(tablef.reshape(B * M, 2 * d + 16), knn_qm)
    res = _tattn(gathered, q.reshape(B * M, d), qpad3.reshape(B * M, 16),
                 f.reshape(B * M, d), p, K, d, bma)
    return res.reshape(B, M, d)


def kernel(pointcloud, params):
    xyz = pointcloud[..., :3]
    feats = pointcloud[..., 3:]
    xyz1, f1, inds1, _ = _sa_module(
        xyz, feats, 2048, 0.04, 64, params['sa1'], bm=64, bmm=512)
    f1 = _transformer(xyz1, f1, params['t1'], 16, bm=128, bmp=256, bma=32)
    xyz2, f2, inds2, fp2_inds = _sa_module(
        xyz1, f1, 1024, 0.1, 32, params['sa2'], bm=128, bmm=256, tbl=inds1)
    f2 = _transformer(xyz2, f2, params['t2'], 16, bm=128, bmp=256, bma=16)
    return jnp.transpose(f2, (0, 2, 1)), xyz2, fp2_inds
